# baseline (speedup 1.0000x reference)
import jax
import jax.numpy as jnp
from jax.experimental import pallas as pl

_VOCAB = 1000
_ROWS = 4096
_COLS = 20
_BLK = 64


def _onehot_block(x_ref, o_ref):
    ids = jax.lax.broadcasted_iota(jnp.int32, (_BLK, _COLS, _VOCAB), 2)
    o_ref[...] = (ids == x_ref[...][:, :, None]).astype(jnp.float32)


def kernel(x):
    return pl.pallas_call(
        _onehot_block,
        grid=(_ROWS // _BLK,),
        in_specs=[pl.BlockSpec((_BLK, _COLS), lambda i: (i, 0))],
        out_specs=pl.BlockSpec((_BLK, _COLS, _VOCAB), lambda i: (i, 0, 0)),
        out_shape=jax.ShapeDtypeStruct((_ROWS, _COLS, _VOCAB), jnp.float32),
    )(x)
